# baseline (speedup 1.0000x reference)
import functools

import jax
import jax.numpy as jnp
from jax import lax
from jax.experimental import pallas as pl
from jax.experimental.pallas import tpu as pltpu
from jax.experimental.pallas import tpu_sc as plsc

_NA = 60
_K = 8
_INPUT_RADIUS = 0.4
_SIGMA = 0.5 * _INPUT_RADIUS ** 2
_N = 1024
_B = 2
_P = _B * _N
_E = _P * _K
_NW = 32
_EPW = _E // _NW
_BIG = 1e30


def _leaky(v):
    return jnp.where(v >= 0, v, 0.01 * v)


def _knn_kernel(x_ref, xt_ref, w0_ref, b0_ref, ft1_ref, idx_ref, wn_ref):
    b = pl.program_id(0)
    x = x_ref[0]
    xt = xt_ref[0]

    d2 = None
    for c in range(3):
        diff = x[:, c:c + 1] - xt[c:c + 1, :]
        sq = diff * diff
        d2 = sq if d2 is None else d2 + sq

    lane = jax.lax.broadcasted_iota(jnp.int32, (_N, _N), 1)
    row = jax.lax.broadcasted_iota(jnp.int32, (_N, _N), 0)
    d2m = jnp.where(lane == row, _BIG, d2)
    idxs = []
    dists = []
    for _ in range(_K):
        m = jnp.min(d2m, axis=1, keepdims=True)
        ik = jnp.min(jnp.where(d2m == m, lane, _N), axis=1, keepdims=True)
        d2m = jnp.where(lane == ik, _BIG, d2m)
        idxs.append(ik)
        dists.append(m)

    ws = [jnp.exp(-d / _SIGMA) for d in dists]
    s0 = functools.reduce(lambda a, bb: a + bb, ws)
    rden = 1.0 / (s0 + 1e-8)
    s = s0 * rden

    idx_ref[0] = jnp.concatenate(idxs, axis=1) + b * _N
    wn_ref[0] = jnp.concatenate(ws, axis=1) * rden
    ft1_ref[...] = _leaky(s * w0_ref[...] + b0_ref[...])


def _sc_gather_body(feat_hbm, idx_hbm, out_hbm, idx_v, rows_v, sem):
    wid = lax.axis_index("s") * 2 + lax.axis_index("c")
    pltpu.sync_copy(idx_hbm.at[wid], idx_v)
    copies = [
        pltpu.async_copy(feat_hbm.at[idx_v.at[j]],
                         rows_v.at[pl.ds(j * 128, 128)], sem)
        for j in range(4)
    ]
    for c in copies:
        c.wait()
    pltpu.sync_copy(rows_v, out_hbm.at[pl.ds(wid * _EPW, _EPW)])


def _make_sc_gather():
    return functools.partial(
        pl.kernel,
        out_type=jax.ShapeDtypeStruct((_E, 32), jnp.float32),
        mesh=plsc.VectorSubcoreMesh(core_axis_name="c", subcore_axis_name="s"),
        scratch_types=[
            pltpu.VMEM((4, 128), jnp.int32),
            pltpu.VMEM((_EPW, 32), jnp.float32),
            pltpu.SemaphoreType.DMA,
        ],
        compiler_params=pltpu.CompilerParams(use_tc_tiling_on_sc=False),
    )(_sc_gather_body)


def _conv_kernel(g_ref, wn_ref, w_ref, b_ref, out_ref):
    agg = None
    for k in range(_K):
        t = wn_ref[:, k:k + 1] * g_ref[:, k, :]
        agg = t if agg is None else agg + t
    out_ref[...] = _leaky(
        jnp.dot(agg, w_ref[...], preferred_element_type=jnp.float32)
        + b_ref[...])


def _out_kernel(feat_ref, w1_ref, b1_ref, w2_ref, b2_ref, aq_ref,
                fcw_ref, fcb_ref, tw_ref, tb_ref, out_ref):
    h = jnp.maximum(
        jnp.dot(feat_ref[...], w1_ref[...], preferred_element_type=jnp.float32)
        + b1_ref[...], 0.0)
    h = jnp.dot(h, w2_ref[...], preferred_element_type=jnp.float32) + b2_ref[...]

    aq = aq_ref[...]
    q = jnp.mean(aq, axis=0, keepdims=True)
    qn = q / (jnp.sqrt(jnp.sum(q * q)) + 1e-8)

    rows = []
    for b in range(_B):
        hb = h[b * _N:(b + 1) * _N]
        gfeat = jnp.max(hb, axis=0, keepdims=True)
        fc = jnp.maximum(
            jnp.dot(gfeat, fcw_ref[...], preferred_element_type=jnp.float32)
            + fcb_ref[...], 0.0)
        t_out = jnp.dot(fc, tw_ref[...], preferred_element_type=jnp.float32) \
            + tb_ref[...]
        rows.append(jnp.concatenate([qn, t_out, fc], axis=1))
    out_ref[...] = jnp.concatenate(rows, axis=0)


@jax.jit
def kernel(x, params):
    xt = jnp.swapaxes(x, 1, 2)
    convs = params["convs"]
    w0 = convs[0][0]
    b0 = convs[0][1].reshape(1, 32)

    fixed = lambda *zeros: (lambda i: zeros)
    ft1, idxg, wn = pl.pallas_call(
        _knn_kernel,
        out_shape=(
            jax.ShapeDtypeStruct((_P, 32), jnp.float32),
            jax.ShapeDtypeStruct((_B, _N, _K), jnp.int32),
            jax.ShapeDtypeStruct((_B, _N, _K), jnp.float32),
        ),
        grid=(_B,),
        in_specs=[
            pl.BlockSpec((1, _N, 3), lambda i: (i, 0, 0)),
            pl.BlockSpec((1, 3, _N), lambda i: (i, 0, 0)),
            pl.BlockSpec((1, 32), fixed(0, 0)),
            pl.BlockSpec((1, 32), fixed(0, 0)),
        ],
        out_specs=(
            pl.BlockSpec((_N, 32), lambda i: (i, 0)),
            pl.BlockSpec((1, _N, _K), lambda i: (i, 0, 0)),
            pl.BlockSpec((1, _N, _K), lambda i: (i, 0, 0)),
        ),
    )(x, xt, w0, b0)

    idx3 = idxg.reshape(_NW, 4, 128)
    wn2 = wn.reshape(_P, _K)

    sc_gather = _make_sc_gather()
    feat = ft1
    for l in range(1, 8):
        g = sc_gather(feat, idx3)
        feat = pl.pallas_call(
            _conv_kernel,
            out_shape=jax.ShapeDtypeStruct((_P, 32), jnp.float32),
        )(g.reshape(_P, _K, 32), wn2, convs[l][0], convs[l][1].reshape(1, 32))

    out = pl.pallas_call(
        _out_kernel,
        out_shape=jax.ShapeDtypeStruct((_B, 71), jnp.float32),
    )(feat,
      params["out_W1"], params["out_b1"].reshape(1, 128),
      params["out_W2"], params["out_b2"].reshape(1, 128),
      params["anchor_quats"],
      params["fc_W"], params["fc_b"].reshape(1, 64),
      params["t_W"], params["t_b"].reshape(1, 3))
    return out


# speedup vs baseline: 4.7330x; 4.7330x over previous
import functools

import jax
import jax.numpy as jnp
from jax.experimental import pallas as pl

_NA = 60
_K = 8
_INPUT_RADIUS = 0.4
_SIGMA = 0.5 * _INPUT_RADIUS ** 2
_N = 1024
_IBIG = 2**31 - 1


def _leaky(v):
    return jnp.where(v >= 0, v, 0.01 * v)


def _fwd_kernel(x_ref, xt_ref, w0_ref, b0_ref, ws_ref, bs_ref,
                w1_ref, b1_ref, w2_ref, b2_ref, aq_ref,
                fcw_ref, fcb_ref, tw_ref, tb_ref, out_ref):
    x = x_ref[0]
    xt = xt_ref[0]

    cross = jnp.dot(x, xt, preferred_element_type=jnp.float32)
    n_col = jnp.sum(x * x, axis=1, keepdims=True)
    n_row = jnp.sum(xt * xt, axis=0, keepdims=True)
    d2 = jnp.maximum(n_col + n_row - 2.0 * cross, 0.0)

    lane = jax.lax.broadcasted_iota(jnp.int32, (_N, _N), 1)
    row = jax.lax.broadcasted_iota(jnp.int32, (_N, _N), 0)
    diag = lane == row

    keys = (jax.lax.bitcast_convert_type(d2, jnp.int32) & ~1023) | lane
    keys = jnp.where(diag, _IBIG, keys)
    for _ in range(_K):
        m = jnp.min(keys, axis=1, keepdims=True)
        keys = jnp.where(keys == m, _IBIG, keys)

    nbr = (keys == _IBIG) & jnp.logical_not(diag)
    wfull = jnp.where(nbr, jnp.exp(d2 * (-1.0 / _SIGMA)), 0.0)
    s0 = jnp.sum(wfull, axis=1, keepdims=True)
    rden = 1.0 / (s0 + 1e-8)
    s = s0 * rden
    m_acc = wfull * rden

    feat = _leaky(s * w0_ref[...] + b0_ref[...])
    for l in range(7):
        agg = jnp.dot(m_acc, feat, preferred_element_type=jnp.float32)
        feat = _leaky(jnp.dot(agg, ws_ref[l], preferred_element_type=jnp.float32)
                      + bs_ref[l:l + 1, :])

    h = jnp.maximum(jnp.dot(feat, w1_ref[...], preferred_element_type=jnp.float32)
                    + b1_ref[...], 0.0)
    h = jnp.dot(h, w2_ref[...], preferred_element_type=jnp.float32) + b2_ref[...]

    gfeat = jnp.max(h, axis=0, keepdims=True)
    fc = jnp.maximum(jnp.dot(gfeat, fcw_ref[...], preferred_element_type=jnp.float32)
                     + fcb_ref[...], 0.0)
    t_out = jnp.dot(fc, tw_ref[...], preferred_element_type=jnp.float32) + tb_ref[...]

    aq = aq_ref[...]
    q = jnp.mean(aq, axis=0, keepdims=True)
    qn = q / (jnp.sqrt(jnp.sum(q * q)) + 1e-8)

    i = pl.program_id(0)
    out_ref[pl.ds(i, 1), :] = jnp.concatenate([qn, t_out, fc], axis=1)


@jax.jit
def kernel(x, params):
    b = x.shape[0]
    xt = jnp.swapaxes(x, 1, 2)
    convs = params["convs"]
    w0 = convs[0][0]
    b0 = convs[0][1].reshape(1, 32)
    ws = jnp.stack([w for w, _ in convs[1:]])
    bs = jnp.stack([bb for _, bb in convs[1:]])

    fixed = lambda *zeros: (lambda i: zeros)
    out = pl.pallas_call(
        _fwd_kernel,
        out_shape=jax.ShapeDtypeStruct((b, 71), jnp.float32),
        grid=(b,),
        in_specs=[
            pl.BlockSpec((1, _N, 3), lambda i: (i, 0, 0)),
            pl.BlockSpec((1, 3, _N), lambda i: (i, 0, 0)),
            pl.BlockSpec((1, 32), fixed(0, 0)),
            pl.BlockSpec((1, 32), fixed(0, 0)),
            pl.BlockSpec((7, 32, 32), fixed(0, 0, 0)),
            pl.BlockSpec((7, 32), fixed(0, 0)),
            pl.BlockSpec((32, 128), fixed(0, 0)),
            pl.BlockSpec((1, 128), fixed(0, 0)),
            pl.BlockSpec((128, 128), fixed(0, 0)),
            pl.BlockSpec((1, 128), fixed(0, 0)),
            pl.BlockSpec((_NA, 4), fixed(0, 0)),
            pl.BlockSpec((128, 64), fixed(0, 0)),
            pl.BlockSpec((1, 64), fixed(0, 0)),
            pl.BlockSpec((64, 3), fixed(0, 0)),
            pl.BlockSpec((1, 3), fixed(0, 0)),
        ],
        out_specs=pl.BlockSpec((b, 71), lambda i: (0, 0)),
    )(x, xt, w0, b0, ws, bs,
      params["out_W1"], params["out_b1"].reshape(1, 128),
      params["out_W2"], params["out_b2"].reshape(1, 128),
      params["anchor_quats"],
      params["fc_W"], params["fc_b"].reshape(1, 64),
      params["t_W"], params["t_b"].reshape(1, 3))
    return out


# speedup vs baseline: 4.7770x; 1.0093x over previous
import jax
import jax.numpy as jnp
from jax.experimental import pallas as pl

_NA = 60
_K = 8
_INPUT_RADIUS = 0.4
_SIGMA = 0.5 * _INPUT_RADIUS ** 2
_N = 1024
_B = 2
_IBIG = 2**31 - 1


def _leaky(v):
    return jnp.where(v >= 0, v, 0.01 * v)


def _fwd_kernel(x_ref, xt_ref, w0_ref, b0_ref, ws_ref, bs_ref,
                w1_ref, b1_ref, w2_ref, b2_ref, aq_ref,
                fcw_ref, fcb_ref, tw_ref, tb_ref, out_ref):
    lane = jax.lax.broadcasted_iota(jnp.int32, (_N, _N), 1)
    row = jax.lax.broadcasted_iota(jnp.int32, (_N, _N), 0)
    diag = lane == row

    d2s = []
    for b in range(_B):
        x = x_ref[b]
        xt = xt_ref[b]
        cross = jnp.dot(x, xt, preferred_element_type=jnp.float32)
        n_col = jnp.sum(x * x, axis=1, keepdims=True)
        n_row = jnp.sum(xt * xt, axis=0, keepdims=True)
        d2s.append(jnp.maximum(n_col + n_row - 2.0 * cross, 0.0))

    keys = [
        jnp.where(diag, _IBIG,
                  (jax.lax.bitcast_convert_type(d2, jnp.int32) & ~1023) | lane)
        for d2 in d2s
    ]
    for _ in range(_K):
        for b in range(_B):
            m = jnp.min(keys[b], axis=1, keepdims=True)
            keys[b] = jnp.where(keys[b] == m, _IBIG, keys[b])

    ms = []
    ss = []
    for b in range(_B):
        nbr = (keys[b] == _IBIG) & jnp.logical_not(diag)
        wfull = jnp.where(nbr, jnp.exp(d2s[b] * (-1.0 / _SIGMA)), 0.0)
        s0 = jnp.sum(wfull, axis=1, keepdims=True)
        rden = 1.0 / (s0 + 1e-8)
        ss.append(s0 * rden)
        ms.append(wfull * rden)

    feats = [_leaky(ss[b] * w0_ref[...] + b0_ref[...]) for b in range(_B)]
    for l in range(7):
        for b in range(_B):
            agg = jnp.dot(ms[b], feats[b], preferred_element_type=jnp.float32)
            feats[b] = _leaky(
                jnp.dot(agg, ws_ref[l], preferred_element_type=jnp.float32)
                + bs_ref[l:l + 1, :])

    aq = aq_ref[...]
    q = jnp.mean(aq, axis=0, keepdims=True)
    qn = q / (jnp.sqrt(jnp.sum(q * q)) + 1e-8)

    rows = []
    for b in range(_B):
        h = jnp.maximum(
            jnp.dot(feats[b], w1_ref[...], preferred_element_type=jnp.float32)
            + b1_ref[...], 0.0)
        h = jnp.dot(h, w2_ref[...], preferred_element_type=jnp.float32) \
            + b2_ref[...]
        gfeat = jnp.max(h, axis=0, keepdims=True)
        fc = jnp.maximum(
            jnp.dot(gfeat, fcw_ref[...], preferred_element_type=jnp.float32)
            + fcb_ref[...], 0.0)
        t_out = jnp.dot(fc, tw_ref[...], preferred_element_type=jnp.float32) \
            + tb_ref[...]
        rows.append(jnp.concatenate([qn, t_out, fc], axis=1))
    out_ref[...] = jnp.concatenate(rows, axis=0)


@jax.jit
def kernel(x, params):
    xt = jnp.swapaxes(x, 1, 2)
    convs = params["convs"]
    w0 = convs[0][0]
    b0 = convs[0][1].reshape(1, 32)
    ws = jnp.stack([w for w, _ in convs[1:]])
    bs = jnp.stack([bb for _, bb in convs[1:]])

    out = pl.pallas_call(
        _fwd_kernel,
        out_shape=jax.ShapeDtypeStruct((_B, 71), jnp.float32),
    )(x, xt, w0, b0, ws, bs,
      params["out_W1"], params["out_b1"].reshape(1, 128),
      params["out_W2"], params["out_b2"].reshape(1, 128),
      params["anchor_quats"],
      params["fc_W"], params["fc_b"].reshape(1, 64),
      params["t_W"], params["t_b"].reshape(1, 3))
    return out


# speedup vs baseline: 5.4841x; 1.1480x over previous
import jax
import jax.numpy as jnp
from jax.experimental import pallas as pl

_NA = 60
_K = 8
_INPUT_RADIUS = 0.4
_SIGMA = 0.5 * _INPUT_RADIUS ** 2
_N = 1024
_B = 2
_IBIG = 2**31 - 1


def _leaky(v):
    return jnp.where(v >= 0, v, 0.01 * v)


def _fwd_kernel(x_ref, w0_ref, b0_ref,
                wl0, wl1, wl2, wl3, wl4, wl5, wl6,
                bl0, bl1, bl2, bl3, bl4, bl5, bl6,
                w1_ref, b1_ref, w2_ref, b2_ref, aq_ref,
                fcw_ref, fcb_ref, tw_ref, tb_ref, out_ref):
    w_l_refs = (wl0, wl1, wl2, wl3, wl4, wl5, wl6)
    b_l_refs = (bl0, bl1, bl2, bl3, bl4, bl5, bl6)
    lane = jax.lax.broadcasted_iota(jnp.int32, (_N, _N), 1)
    row = jax.lax.broadcasted_iota(jnp.int32, (_N, _N), 0)
    diag = lane == row

    d2s = []
    for b in range(_B):
        x = x_ref[b]
        xt = jnp.transpose(x)
        cross = jnp.dot(x, xt, preferred_element_type=jnp.float32)
        n_col = jnp.sum(x * x, axis=1, keepdims=True)
        n_row = jnp.transpose(n_col)
        d2s.append(jnp.maximum(n_col + n_row - 2.0 * cross, 0.0))

    keys = [
        jnp.where(diag, _IBIG,
                  (jax.lax.bitcast_convert_type(d2, jnp.int32) & ~1023) | lane)
        for d2 in d2s
    ]
    for _ in range(_K):
        for b in range(_B):
            m = jnp.min(keys[b], axis=1, keepdims=True)
            keys[b] = jnp.where(keys[b] == m, _IBIG, keys[b])

    ms = []
    ss = []
    for b in range(_B):
        nbr = (keys[b] == _IBIG) & jnp.logical_not(diag)
        wfull = jnp.where(nbr, jnp.exp(d2s[b] * (-1.0 / _SIGMA)), 0.0)
        s0 = jnp.sum(wfull, axis=1, keepdims=True)
        rden = 1.0 / (s0 + 1e-8)
        ss.append(s0 * rden)
        ms.append(wfull * rden)

    feats = [_leaky(ss[b] * w0_ref[...] + b0_ref[...][None, :])
             for b in range(_B)]
    for l in range(7):
        for b in range(_B):
            agg = jnp.dot(ms[b], feats[b], preferred_element_type=jnp.float32)
            feats[b] = _leaky(
                jnp.dot(agg, w_l_refs[l][...], preferred_element_type=jnp.float32)
                + b_l_refs[l][...][None, :])

    aq = aq_ref[...]
    q = jnp.mean(aq, axis=0, keepdims=True)
    qn = q / (jnp.sqrt(jnp.sum(q * q)) + 1e-8)

    rows = []
    for b in range(_B):
        h = jnp.maximum(
            jnp.dot(feats[b], w1_ref[...], preferred_element_type=jnp.float32)
            + b1_ref[...][None, :], 0.0)
        h = jnp.dot(h, w2_ref[...], preferred_element_type=jnp.float32) \
            + b2_ref[...][None, :]
        gfeat = jnp.max(h, axis=0, keepdims=True)
        fc = jnp.maximum(
            jnp.dot(gfeat, fcw_ref[...], preferred_element_type=jnp.float32)
            + fcb_ref[...][None, :], 0.0)
        t_out = jnp.dot(fc, tw_ref[...], preferred_element_type=jnp.float32) \
            + tb_ref[...][None, :]
        rows.append(jnp.concatenate([qn, t_out, fc], axis=1))
    out_ref[...] = jnp.concatenate(rows, axis=0)


@jax.jit
def kernel(x, params):
    convs = params["convs"]
    out = pl.pallas_call(
        _fwd_kernel,
        out_shape=jax.ShapeDtypeStruct((_B, 71), jnp.float32),
    )(x, convs[0][0], convs[0][1],
      *[w for w, _ in convs[1:]], *[bb for _, bb in convs[1:]],
      params["out_W1"], params["out_b1"],
      params["out_W2"], params["out_b2"],
      params["anchor_quats"],
      params["fc_W"], params["fc_b"],
      params["t_W"], params["t_b"])
    return out


# speedup vs baseline: 7.0415x; 1.2840x over previous
import jax
import jax.numpy as jnp
from jax.experimental import pallas as pl

_NA = 60
_K = 8
_INPUT_RADIUS = 0.4
_SIGMA = 0.5 * _INPUT_RADIUS ** 2
_N = 1024
_B = 2
_IBIG = 2**31 - 1


def _leaky(v):
    return jnp.where(v >= 0, v, 0.01 * v)


def _fwd_kernel(x_ref, w0_ref, b0_ref,
                wl0, wl1, wl2, wl3, wl4, wl5, wl6,
                bl0, bl1, bl2, bl3, bl4, bl5, bl6,
                w1_ref, b1_ref, w2_ref, b2_ref, aq_ref,
                fcw_ref, fcb_ref, tw_ref, tb_ref, out_ref):
    w_l_refs = (wl0, wl1, wl2, wl3, wl4, wl5, wl6)
    b_l_refs = (bl0, bl1, bl2, bl3, bl4, bl5, bl6)
    lane = jax.lax.broadcasted_iota(jnp.int32, (_N, _N), 1)
    row = jax.lax.broadcasted_iota(jnp.int32, (_N, _N), 0)
    diag = lane == row

    d2s = []
    for b in range(_B):
        x = x_ref[b]
        xt = jnp.transpose(x)
        cross = jnp.dot(x, xt, preferred_element_type=jnp.float32)
        n_col = jnp.sum(x * x, axis=1, keepdims=True)
        n_row = jnp.transpose(n_col)
        d2s.append(jnp.maximum(n_col + n_row - 2.0 * cross, 0.0))

    keys = [
        jnp.where(diag, _IBIG,
                  (jax.lax.bitcast_convert_type(d2, jnp.int32) & ~1023) | row)
        for d2 in d2s
    ]
    for _ in range(_K):
        for b in range(_B):
            m = jnp.min(keys[b], axis=0, keepdims=True)
            keys[b] = jnp.where(keys[b] == m, _IBIG, keys[b])

    mts = []
    ss = []
    for b in range(_B):
        nbr = (keys[b] == _IBIG) & jnp.logical_not(diag)
        wfull = jnp.where(nbr, jnp.exp(d2s[b] * (-1.0 / _SIGMA)), 0.0)
        s0 = jnp.sum(wfull, axis=0, keepdims=True)
        rden = 1.0 / (s0 + 1e-8)
        ss.append(jnp.transpose(s0 * rden))
        mts.append(wfull * rden)

    dnt = (((0,), (0,)), ((), ()))
    feats = [_leaky(ss[b] * w0_ref[...] + b0_ref[...][None, :])
             for b in range(_B)]
    for l in range(7):
        for b in range(_B):
            agg_t = jax.lax.dot_general(
                feats[b], mts[b], dnt, preferred_element_type=jnp.float32)
            feats[b] = _leaky(
                jax.lax.dot_general(agg_t, w_l_refs[l][...], dnt,
                                    preferred_element_type=jnp.float32)
                + b_l_refs[l][...][None, :])

    aq = aq_ref[...]
    q = jnp.mean(aq, axis=0, keepdims=True)
    qn = q / (jnp.sqrt(jnp.sum(q * q)) + 1e-8)

    rows = []
    for b in range(_B):
        h = jnp.maximum(
            jnp.dot(feats[b], w1_ref[...], preferred_element_type=jnp.float32)
            + b1_ref[...][None, :], 0.0)
        h = jnp.dot(h, w2_ref[...], preferred_element_type=jnp.float32) \
            + b2_ref[...][None, :]
        gfeat = jnp.max(h, axis=0, keepdims=True)
        fc = jnp.maximum(
            jnp.dot(gfeat, fcw_ref[...], preferred_element_type=jnp.float32)
            + fcb_ref[...][None, :], 0.0)
        t_out = jnp.dot(fc, tw_ref[...], preferred_element_type=jnp.float32) \
            + tb_ref[...][None, :]
        rows.append(jnp.concatenate([qn, t_out, fc], axis=1))
    out_ref[...] = jnp.concatenate(rows, axis=0)


@jax.jit
def kernel(x, params):
    convs = params["convs"]
    out = pl.pallas_call(
        _fwd_kernel,
        out_shape=jax.ShapeDtypeStruct((_B, 71), jnp.float32),
    )(x, convs[0][0], convs[0][1],
      *[w for w, _ in convs[1:]], *[bb for _, bb in convs[1:]],
      params["out_W1"], params["out_b1"],
      params["out_W2"], params["out_b2"],
      params["anchor_quats"],
      params["fc_W"], params["fc_b"],
      params["t_W"], params["t_b"])
    return out


# speedup vs baseline: 7.2138x; 1.0245x over previous
import jax
import jax.numpy as jnp
from jax.experimental import pallas as pl

_NA = 60
_K = 8
_INPUT_RADIUS = 0.4
_SIGMA = 0.5 * _INPUT_RADIUS ** 2
_N = 1024
_B = 2
_IBIG = 2**31 - 1


def _leaky(v):
    return jnp.where(v >= 0, v, 0.01 * v)


def _fwd_kernel(x_ref, w0_ref, b0_ref,
                wl0, wl1, wl2, wl3, wl4, wl5, wl6,
                bl0, bl1, bl2, bl3, bl4, bl5, bl6,
                w1_ref, b1_ref, w2_ref, b2_ref, aq_ref,
                fcw_ref, fcb_ref, tw_ref, tb_ref, out_ref):
    w_l_refs = (wl0, wl1, wl2, wl3, wl4, wl5, wl6)
    b_l_refs = (bl0, bl1, bl2, bl3, bl4, bl5, bl6)
    lane = jax.lax.broadcasted_iota(jnp.int32, (_N, _N), 1)
    row = jax.lax.broadcasted_iota(jnp.int32, (_N, _N), 0)
    diag = lane == row

    d2s = []
    for b in range(_B):
        x = x_ref[b]
        xt = jnp.transpose(x)
        cross = jnp.dot(x, xt, preferred_element_type=jnp.float32)
        n_col = jnp.sum(x * x, axis=1, keepdims=True)
        n_row = jnp.transpose(n_col)
        d2s.append(jnp.maximum(n_col + n_row - 2.0 * cross, 0.0))

    keys = [
        jnp.where(diag, _IBIG,
                  (jax.lax.bitcast_convert_type(d2, jnp.int32) & ~1023) | row)
        for d2 in d2s
    ]
    kth = [None] * _B
    for p in range(_K):
        for b in range(_B):
            cand = (keys[b] if p == 0
                    else jnp.where(keys[b] > kth[b], keys[b], _IBIG))
            kth[b] = jnp.min(cand, axis=0, keepdims=True)

    mts = []
    ss = []
    for b in range(_B):
        nbr = (keys[b] <= kth[b]) & jnp.logical_not(diag)
        wfull = jnp.where(nbr, jnp.exp(d2s[b] * (-1.0 / _SIGMA)), 0.0)
        s0 = jnp.sum(wfull, axis=0, keepdims=True)
        rden = 1.0 / (s0 + 1e-8)
        ss.append(jnp.transpose(s0 * rden))
        mts.append(wfull * rden)

    dnt = (((0,), (0,)), ((), ()))
    feats = [_leaky(ss[b] * w0_ref[...] + b0_ref[...][None, :])
             for b in range(_B)]
    for l in range(7):
        for b in range(_B):
            agg_t = jax.lax.dot_general(
                feats[b], mts[b], dnt, preferred_element_type=jnp.float32)
            feats[b] = _leaky(
                jax.lax.dot_general(agg_t, w_l_refs[l][...], dnt,
                                    preferred_element_type=jnp.float32)
                + b_l_refs[l][...][None, :])

    aq = aq_ref[...]
    q = jnp.mean(aq, axis=0, keepdims=True)
    qn = q / (jnp.sqrt(jnp.sum(q * q)) + 1e-8)

    rows = []
    for b in range(_B):
        h = jnp.maximum(
            jnp.dot(feats[b], w1_ref[...], preferred_element_type=jnp.float32)
            + b1_ref[...][None, :], 0.0)
        h = jnp.dot(h, w2_ref[...], preferred_element_type=jnp.float32) \
            + b2_ref[...][None, :]
        gfeat = jnp.max(h, axis=0, keepdims=True)
        fc = jnp.maximum(
            jnp.dot(gfeat, fcw_ref[...], preferred_element_type=jnp.float32)
            + fcb_ref[...][None, :], 0.0)
        t_out = jnp.dot(fc, tw_ref[...], preferred_element_type=jnp.float32) \
            + tb_ref[...][None, :]
        rows.append(jnp.concatenate([qn, t_out, fc], axis=1))
    out_ref[...] = jnp.concatenate(rows, axis=0)


@jax.jit
def kernel(x, params):
    convs = params["convs"]
    out = pl.pallas_call(
        _fwd_kernel,
        out_shape=jax.ShapeDtypeStruct((_B, 71), jnp.float32),
    )(x, convs[0][0], convs[0][1],
      *[w for w, _ in convs[1:]], *[bb for _, bb in convs[1:]],
      params["out_W1"], params["out_b1"],
      params["out_W2"], params["out_b2"],
      params["anchor_quats"],
      params["fc_W"], params["fc_b"],
      params["t_W"], params["t_b"])
    return out


# speedup vs baseline: 7.2148x; 1.0001x over previous
import jax
import jax.numpy as jnp
from jax.experimental import pallas as pl

_K = 8
_INPUT_RADIUS = 0.4
_SIGMA = 0.5 * _INPUT_RADIUS ** 2
_N = 1024
_B = 2
_IBIG = 2**31 - 1


def _leaky(v):
    return jnp.where(v >= 0, v, 0.01 * v)


def _fwd_kernel(x_ref, w0_ref, b0_ref,
                wl0, wl1, wl2, wl3, wl4, wl5, wl6,
                bl0, bl1, bl2, bl3, bl4, bl5, bl6,
                w1_ref, b1_ref, w2_ref, b2_ref, aq_ref,
                fcw_ref, fcb_ref, tw_ref, tb_ref, out_ref):
    w_l_refs = (wl0, wl1, wl2, wl3, wl4, wl5, wl6)
    b_l_refs = (bl0, bl1, bl2, bl3, bl4, bl5, bl6)
    lane = jax.lax.broadcasted_iota(jnp.int32, (_N, _N), 1)
    row = jax.lax.broadcasted_iota(jnp.int32, (_N, _N), 0)
    diag = lane == row

    d2s = []
    for b in range(_B):
        x = x_ref[b]
        xt = jnp.transpose(x)
        cross = jnp.dot(x, xt, preferred_element_type=jnp.float32)
        n_col = jnp.sum(x * x, axis=1, keepdims=True)
        n_row = jnp.transpose(n_col)
        d2s.append(jnp.maximum(n_col + n_row - 2.0 * cross, 0.0))

    keys = [
        jnp.where(diag, _IBIG,
                  (jax.lax.bitcast_convert_type(d2, jnp.int32) & ~1023) | row)
        for d2 in d2s
    ]
    kth = [None] * _B
    for p in range(_K):
        for b in range(_B):
            cand = (keys[b] if p == 0
                    else jnp.where(keys[b] > kth[b], keys[b], _IBIG))
            kth[b] = jnp.min(cand, axis=0, keepdims=True)

    mts = []
    ss = []
    for b in range(_B):
        nbr = (keys[b] <= kth[b]) & jnp.logical_not(diag)
        wfull = jnp.where(nbr, jnp.exp(d2s[b] * (-1.0 / _SIGMA)), 0.0)
        s0 = jnp.sum(wfull, axis=0, keepdims=True)
        rden = 1.0 / (s0 + 1e-8)
        ss.append(jnp.transpose(s0 * rden))
        mts.append(wfull * rden)

    dnt = (((0,), (0,)), ((), ()))
    feats = [_leaky(ss[b] * w0_ref[...] + b0_ref[...][None, :])
             for b in range(_B)]
    for l in range(7):
        for b in range(_B):
            agg_t = jax.lax.dot_general(
                feats[b], mts[b], dnt, preferred_element_type=jnp.float32)
            feats[b] = _leaky(
                jax.lax.dot_general(agg_t, w_l_refs[l][...], dnt,
                                    preferred_element_type=jnp.float32)
                + b_l_refs[l][...][None, :])

    aq = aq_ref[...]
    q = jnp.mean(aq, axis=0, keepdims=True)
    qn = q / (jnp.sqrt(jnp.sum(q * q)) + 1e-8)

    rows = []
    for b in range(_B):
        h = jnp.maximum(
            jnp.dot(feats[b], w1_ref[...], preferred_element_type=jnp.float32)
            + b1_ref[...][None, :], 0.0)
        h = jnp.dot(h, w2_ref[...], preferred_element_type=jnp.float32) \
            + b2_ref[...][None, :]
        gfeat = jnp.max(h, axis=0, keepdims=True)
        fc = jnp.maximum(
            jnp.dot(gfeat, fcw_ref[...], preferred_element_type=jnp.float32)
            + fcb_ref[...][None, :], 0.0)
        t_out = jnp.dot(fc, tw_ref[...], preferred_element_type=jnp.float32) \
            + tb_ref[...][None, :]
        rows.append(jnp.concatenate([qn, t_out, fc], axis=1))
    out_ref[...] = jnp.concatenate(rows, axis=0)


@jax.jit
def kernel(x, params):
    convs = params["convs"]
    out = pl.pallas_call(
        _fwd_kernel,
        out_shape=jax.ShapeDtypeStruct((_B, 71), jnp.float32),
    )(x, convs[0][0], convs[0][1],
      *[w for w, _ in convs[1:]], *[bb for _, bb in convs[1:]],
      params["out_W1"], params["out_b1"],
      params["out_W2"], params["out_b2"],
      params["anchor_quats"],
      params["fc_W"], params["fc_b"],
      params["t_W"], params["t_b"])
    return out
